# Initial kernel scaffold; baseline (speedup 1.0000x reference)
import functools

import jax
import jax.numpy as jnp
from jax import lax
from jax.experimental import pallas as pl
from jax.experimental.pallas import tpu as pltpu
from jax.experimental.pallas import tpu_sc as plsc

N = 10000
E = 320000
D = 128
C = 128

ROW_BLK = 1024
N_PAD = 10240

NC = 2
NS = 16
NW = NC * NS
CHUNK = 128
NCHUNK = 79
EPW = NCHUNK * CHUNK
E_PAD = NW * EPW
NACC = 10240
ZSTRIPE = NACC // NS


def _err_node_body(h_ref, t_ref, w_ref, b_ref, out_ref):
    y = jnp.dot(h_ref[...], w_ref[...], preferred_element_type=jnp.float32)
    d = y + b_ref[...] - t_ref[...]
    e = jnp.mean(d * d, axis=1)
    out_ref[...] = e.reshape(ROW_BLK // 128, 128)


def _err_node(H, target, W, b):
    grid = (N_PAD // ROW_BLK,)
    return pl.pallas_call(
        _err_node_body,
        grid=grid,
        in_specs=[
            pl.BlockSpec((ROW_BLK, D), lambda g: (g, 0)),
            pl.BlockSpec((ROW_BLK, D), lambda g: (g, 0)),
            pl.BlockSpec((D, D), lambda g: (0, 0)),
            pl.BlockSpec((1, D), lambda g: (0, 0)),
        ],
        out_specs=pl.BlockSpec((ROW_BLK // 128, 128), lambda g: (g, 0)),
        out_shape=jax.ShapeDtypeStruct((N_PAD // 128, 128), jnp.float32),
    )(H, target, W, b.reshape(1, D))


def _sc_scatter_body(src_hbm, dst_hbm, err_hbm, centers_hbm, out_hbm,
                     src_v, dst_v, err_v, vals_v, ones_v, zeros_v,
                     sum_sh, cnt_sh, cen_v, obuf_v, sem_a, sem_b):
    cid = lax.axis_index("c")
    sid = lax.axis_index("s")
    wid = sid * NC + cid

    pltpu.sync_copy(src_hbm.at[wid], src_v)
    pltpu.sync_copy(dst_hbm.at[wid], dst_v)
    pltpu.sync_copy(err_hbm, err_v)

    zero16 = jnp.zeros((16,), jnp.float32)
    for k in range(CHUNK // 16):
        ones_v[pl.ds(k * 16, 16)] = zero16 + 1.0
    for k in range(ZSTRIPE // 16):
        zeros_v[pl.ds(k * 16, 16)] = zero16

    pltpu.sync_copy(zeros_v, sum_sh.at[pl.ds(sid * ZSTRIPE, ZSTRIPE)])
    pltpu.sync_copy(zeros_v, cnt_sh.at[pl.ds(sid * ZSTRIPE, ZSTRIPE)])
    plsc.subcore_barrier()

    def gather_body(i, carry):
        d16 = dst_v[pl.ds(i * 16, 16)]
        vals_v[pl.ds(i * 16, 16)] = plsc.load_gather(err_v, [d16])
        return carry

    lax.fori_loop(0, EPW // 16, gather_body, 0)

    def scatter_body(j, carry):
        a = pltpu.async_copy(
            vals_v.at[pl.ds(j * CHUNK, CHUNK)], sum_sh.at[src_v.at[j]],
            sem_a, add=True)
        c = pltpu.async_copy(ones_v, cnt_sh.at[src_v.at[j]], sem_b, add=True)
        a.wait()
        c.wait()
        return carry

    lax.fori_loop(0, NCHUNK, scatter_body, 0)
    plsc.subcore_barrier()

    @pl.when(sid == 0)
    def _():
        pltpu.sync_copy(centers_hbm, cen_v)
        pltpu.async_copy(sum_sh.at[cen_v], obuf_v, sem_a).wait()
        pltpu.sync_copy(obuf_v, out_hbm.at[cid, 0])
        pltpu.async_copy(cnt_sh.at[cen_v], obuf_v, sem_a).wait()
        pltpu.sync_copy(obuf_v, out_hbm.at[cid, 1])


_sc_scatter = functools.partial(
    pl.kernel,
    _sc_scatter_body,
    out_type=jax.ShapeDtypeStruct((NC, 2, C), jnp.float32),
    mesh=plsc.VectorSubcoreMesh(core_axis_name="c", subcore_axis_name="s"),
    scratch_types=[
        pltpu.VMEM((NCHUNK, CHUNK), jnp.int32),
        pltpu.VMEM((EPW,), jnp.int32),
        pltpu.VMEM((N_PAD,), jnp.float32),
        pltpu.VMEM((EPW,), jnp.float32),
        pltpu.VMEM((CHUNK,), jnp.float32),
        pltpu.VMEM((ZSTRIPE,), jnp.float32),
        pltpu.VMEM_SHARED((NACC,), jnp.float32),
        pltpu.VMEM_SHARED((NACC,), jnp.float32),
        pltpu.VMEM((C,), jnp.int32),
        pltpu.VMEM((C,), jnp.float32),
        pltpu.SemaphoreType.DMA,
        pltpu.SemaphoreType.DMA,
    ],
)


def _final_body(p_ref, out_ref):
    p = p_ref[...]
    loss_sum = p[0:1, :] + p[2:3, :]
    cnt = p[1:2, :] + p[3:4, :]
    aux = jnp.sum(loss_sum / jnp.maximum(cnt, 1.0)) * (1.0 / C)
    pairs = jnp.sum(cnt)
    mpl = jnp.sum(loss_sum) / pairs
    mdeg = jnp.max(cnt)
    lane = lax.broadcasted_iota(jnp.int32, (1, C), 1)
    row = jnp.where(lane == 0, aux,
                    jnp.where(lane == 1, pairs,
                              jnp.where(lane == 2, mpl, mdeg)))
    out_ref[...] = row


def _finalize(partials):
    return pl.pallas_call(
        _final_body,
        out_shape=jax.ShapeDtypeStruct((1, C), jnp.float32),
    )(partials.reshape(2 * 2, C))


def kernel(H, edge_index, centers, target, W, b):
    H = H.astype(jnp.float32)
    target = target.astype(jnp.float32)
    W = W.astype(jnp.float32)
    b = b.astype(jnp.float32)
    edges = edge_index.astype(jnp.int32)
    centers = centers.astype(jnp.int32)

    err2d = _err_node(H, target, W, b)
    err_flat = err2d.reshape(N_PAD)

    src = edges[0]
    dst = edges[1]
    pad = E_PAD - E
    src_p = jnp.concatenate([src, jnp.full((pad,), N, jnp.int32)])
    dst_p = jnp.concatenate([dst, jnp.zeros((pad,), jnp.int32)])
    src3 = src_p.reshape(NW, NCHUNK, CHUNK)
    dst2 = dst_p.reshape(NW, EPW)

    partials = _sc_scatter()(src3, dst2, err_flat, centers)
    row = _finalize(partials)

    aux_loss = row[0, 0]
    stats_pairs = row[0, 1]
    mean_pair_loss = row[0, 2]
    max_deg = row[0, 3]
    stats_centers = jnp.asarray(float(C), dtype=jnp.float32)
    return (aux_loss, stats_centers, stats_pairs, mean_pair_loss, max_deg)


# speedup vs baseline: 23.0849x; 23.0849x over previous
import functools

import jax
import jax.numpy as jnp
from jax import lax
from jax.experimental import pallas as pl
from jax.experimental.pallas import tpu as pltpu
from jax.experimental.pallas import tpu_sc as plsc

N = 10000
E = 320000
D = 128
C = 128

ROW_BLK = 1024
N_PAD = 10240

NC = 2
NS = 16
NW = NC * NS
CHUNK = 128
NCHUNK = 79
EPW = NCHUNK * CHUNK
E_PAD = NW * EPW
NACC = 10240
ZSTRIPE = NACC // NS


def _err_node_body(h_ref, t_ref, w_ref, b_ref, out_ref):
    y = jnp.dot(h_ref[...], w_ref[...], preferred_element_type=jnp.float32)
    d = y + b_ref[...] - t_ref[...]
    e = jnp.mean(d * d, axis=1)
    out_ref[...] = e.reshape(ROW_BLK // 128, 128)


def _err_node(H, target, W, b):
    grid = (N_PAD // ROW_BLK,)
    return pl.pallas_call(
        _err_node_body,
        grid=grid,
        in_specs=[
            pl.BlockSpec((ROW_BLK, D), lambda g: (g, 0)),
            pl.BlockSpec((ROW_BLK, D), lambda g: (g, 0)),
            pl.BlockSpec((D, D), lambda g: (0, 0)),
            pl.BlockSpec((1, D), lambda g: (0, 0)),
        ],
        out_specs=pl.BlockSpec((ROW_BLK // 128, 128), lambda g: (g, 0)),
        out_shape=jax.ShapeDtypeStruct((N_PAD // 128, 128), jnp.float32),
    )(H, target, W, b.reshape(1, D))


def _sc_scatter_body(src_hbm, dst_hbm, err_hbm, centers_hbm, out_hbm,
                     src_v, dst_v, err_v, vals_v, ones_v, zeros_v,
                     sum_sh, cnt_sh, cen_v, obuf_v, sem_a, sem_b):
    cid = lax.axis_index("c")
    sid = lax.axis_index("s")
    wid = sid * NC + cid

    pltpu.sync_copy(src_hbm.at[wid], src_v)
    pltpu.sync_copy(dst_hbm.at[wid], dst_v)
    pltpu.sync_copy(err_hbm, err_v)

    zero16 = jnp.zeros((16,), jnp.float32)
    for k in range(CHUNK // 16):
        ones_v[pl.ds(k * 16, 16)] = zero16 + 1.0
    for k in range(ZSTRIPE // 16):
        zeros_v[pl.ds(k * 16, 16)] = zero16

    pltpu.sync_copy(zeros_v, sum_sh.at[pl.ds(sid * ZSTRIPE, ZSTRIPE)])
    pltpu.sync_copy(zeros_v, cnt_sh.at[pl.ds(sid * ZSTRIPE, ZSTRIPE)])
    plsc.subcore_barrier()

    def gather_body(i, carry):
        d16 = dst_v[pl.ds(i * 16, 16)]
        vals_v[pl.ds(i * 16, 16)] = plsc.load_gather(err_v, [d16])
        return carry

    lax.fori_loop(0, EPW // 16, gather_body, 0)

    def scatter_body(j, carry):
        a = pltpu.async_copy(
            vals_v.at[pl.ds(j * CHUNK, CHUNK)], sum_sh.at[src_v.at[j]],
            sem_a, add=True)
        c = pltpu.async_copy(ones_v, cnt_sh.at[src_v.at[j]], sem_b, add=True)
        a.wait()
        c.wait()
        return carry

    lax.fori_loop(0, NCHUNK, scatter_body, 0)
    plsc.subcore_barrier()

    @pl.when(sid == 0)
    def _():
        pltpu.sync_copy(centers_hbm, cen_v)
        pltpu.async_copy(sum_sh.at[cen_v], obuf_v, sem_a).wait()
        pltpu.sync_copy(obuf_v, out_hbm.at[cid, 0])
        pltpu.async_copy(cnt_sh.at[cen_v], obuf_v, sem_a).wait()
        pltpu.sync_copy(obuf_v, out_hbm.at[cid, 1])


_sc_scatter = functools.partial(
    pl.kernel,
    _sc_scatter_body,
    out_type=jax.ShapeDtypeStruct((NC, 2, C), jnp.float32),
    mesh=plsc.VectorSubcoreMesh(core_axis_name="c", subcore_axis_name="s"),
    compiler_params=pltpu.CompilerParams(needs_layout_passes=False),
    scratch_types=[
        pltpu.VMEM((NCHUNK, CHUNK), jnp.int32),
        pltpu.VMEM((EPW,), jnp.int32),
        pltpu.VMEM((N_PAD,), jnp.float32),
        pltpu.VMEM((EPW,), jnp.float32),
        pltpu.VMEM((CHUNK,), jnp.float32),
        pltpu.VMEM((ZSTRIPE,), jnp.float32),
        pltpu.VMEM_SHARED((NACC,), jnp.float32),
        pltpu.VMEM_SHARED((NACC,), jnp.float32),
        pltpu.VMEM((C,), jnp.int32),
        pltpu.VMEM((C,), jnp.float32),
        pltpu.SemaphoreType.DMA,
        pltpu.SemaphoreType.DMA,
    ],
)


def _final_body(p_ref, out_ref):
    p = p_ref[...]
    loss_sum = p[0:1, :] + p[2:3, :]
    cnt = p[1:2, :] + p[3:4, :]
    aux = jnp.sum(loss_sum / jnp.maximum(cnt, 1.0)) * (1.0 / C)
    pairs = jnp.sum(cnt)
    mpl = jnp.sum(loss_sum) / pairs
    mdeg = jnp.max(cnt)
    lane = lax.broadcasted_iota(jnp.int32, (1, C), 1)
    row = jnp.where(lane == 0, aux,
                    jnp.where(lane == 1, pairs,
                              jnp.where(lane == 2, mpl, mdeg)))
    out_ref[...] = row


def _finalize(partials):
    return pl.pallas_call(
        _final_body,
        out_shape=jax.ShapeDtypeStruct((1, C), jnp.float32),
    )(partials.reshape(2 * 2, C))


def kernel(H, edge_index, centers, target, W, b):
    H = H.astype(jnp.float32)
    target = target.astype(jnp.float32)
    W = W.astype(jnp.float32)
    b = b.astype(jnp.float32)
    edges = edge_index.astype(jnp.int32)
    centers = centers.astype(jnp.int32)

    err2d = _err_node(H, target, W, b)
    err_flat = err2d.reshape(N_PAD)

    src = edges[0]
    dst = edges[1]
    pad = E_PAD - E
    src_p = jnp.concatenate([src, jnp.full((pad,), N, jnp.int32)])
    dst_p = jnp.concatenate([dst, jnp.zeros((pad,), jnp.int32)])
    src3 = src_p.reshape(NW, NCHUNK, CHUNK)
    dst2 = dst_p.reshape(NW, EPW)

    partials = _sc_scatter()(src3, dst2, err_flat, centers)
    row = _finalize(partials)

    aux_loss = row[0, 0]
    stats_pairs = row[0, 1]
    mean_pair_loss = row[0, 2]
    max_deg = row[0, 3]
    stats_centers = jnp.asarray(float(C), dtype=jnp.float32)
    return (aux_loss, stats_centers, stats_pairs, mean_pair_loss, max_deg)


# speedup vs baseline: 26.0903x; 1.1302x over previous
import functools

import jax
import jax.numpy as jnp
from jax import lax
from jax.experimental import pallas as pl
from jax.experimental.pallas import tpu as pltpu
from jax.experimental.pallas import tpu_sc as plsc

N = 10000
E = 320000
D = 128
C = 128

ROW_BLK = 1024
N_PAD = 10240

NC = 2
NS = 16
NW = NC * NS
CHUNK = 128
NCHUNK = 79
EPW = NCHUNK * CHUNK
E_PAD = NW * EPW
NACC = 10240
ZSTRIPE = NACC // NS


def _err_node_body(h_ref, t_ref, w_ref, b_ref, out_ref):
    y = jnp.dot(h_ref[...], w_ref[...], preferred_element_type=jnp.float32)
    d = y + b_ref[...] - t_ref[...]
    e = jnp.mean(d * d, axis=1)
    out_ref[...] = e.reshape(ROW_BLK // 128, 128)


def _err_node(H, target, W, b):
    grid = (N_PAD // ROW_BLK,)
    return pl.pallas_call(
        _err_node_body,
        grid=grid,
        in_specs=[
            pl.BlockSpec((ROW_BLK, D), lambda g: (g, 0)),
            pl.BlockSpec((ROW_BLK, D), lambda g: (g, 0)),
            pl.BlockSpec((D, D), lambda g: (0, 0)),
            pl.BlockSpec((1, D), lambda g: (0, 0)),
        ],
        out_specs=pl.BlockSpec((ROW_BLK // 128, 128), lambda g: (g, 0)),
        out_shape=jax.ShapeDtypeStruct((N_PAD // 128, 128), jnp.float32),
    )(H, target, W, b.reshape(1, D))


def _sc_scatter_body(src_hbm, dst_hbm, err_hbm, centers_hbm, out_hbm,
                     src_v, dst_v, err_v, vals_v, ones_v, zeros_v,
                     sum_sh, cnt_sh, cen_v, obuf_v, sem_a, sem_b):
    cid = lax.axis_index("c")
    sid = lax.axis_index("s")
    wid = sid * NC + cid

    in_src = pltpu.async_copy(src_hbm.at[wid], src_v, sem_a)
    in_dst = pltpu.async_copy(dst_hbm.at[wid], dst_v, sem_a)
    in_err = pltpu.async_copy(err_hbm, err_v, sem_a)

    zero16 = jnp.zeros((16,), jnp.float32)
    for k in range(CHUNK // 16):
        ones_v[pl.ds(k * 16, 16)] = zero16 + 1.0
    for k in range(ZSTRIPE // 16):
        zeros_v[pl.ds(k * 16, 16)] = zero16

    pltpu.sync_copy(zeros_v, sum_sh.at[pl.ds(sid * ZSTRIPE, ZSTRIPE)])
    pltpu.sync_copy(zeros_v, cnt_sh.at[pl.ds(sid * ZSTRIPE, ZSTRIPE)])
    in_src.wait()
    in_dst.wait()
    in_err.wait()
    plsc.subcore_barrier()

    def chunk_body(j, carry):
        base = j * CHUNK
        for k in range(CHUNK // 16):
            d16 = dst_v[pl.ds(base + k * 16, 16)]
            vals_v[pl.ds(base + k * 16, 16)] = plsc.load_gather(err_v, [d16])
        pltpu.async_copy(vals_v.at[pl.ds(base, CHUNK)],
                         sum_sh.at[src_v.at[j]], sem_a, add=True)
        pltpu.async_copy(ones_v, cnt_sh.at[src_v.at[j]], sem_b, add=True)
        return carry

    lax.fori_loop(0, NCHUNK, chunk_body, 0)

    pltpu.make_async_copy(err_hbm.at[pl.ds(0, EPW)], vals_v, sem_a).wait()
    pltpu.make_async_copy(err_hbm.at[pl.ds(0, EPW)], vals_v, sem_b).wait()
    plsc.subcore_barrier()

    @pl.when(sid == 0)
    def _():
        pltpu.sync_copy(centers_hbm, cen_v)
        pltpu.async_copy(sum_sh.at[cen_v], obuf_v, sem_a).wait()
        pltpu.sync_copy(obuf_v, out_hbm.at[cid, 0])
        pltpu.async_copy(cnt_sh.at[cen_v], obuf_v, sem_a).wait()
        pltpu.sync_copy(obuf_v, out_hbm.at[cid, 1])


_sc_scatter = functools.partial(
    pl.kernel,
    _sc_scatter_body,
    out_type=jax.ShapeDtypeStruct((NC, 2, C), jnp.float32),
    mesh=plsc.VectorSubcoreMesh(core_axis_name="c", subcore_axis_name="s"),
    compiler_params=pltpu.CompilerParams(needs_layout_passes=False),
    scratch_types=[
        pltpu.VMEM((NCHUNK, CHUNK), jnp.int32),
        pltpu.VMEM((EPW,), jnp.int32),
        pltpu.VMEM((N_PAD,), jnp.float32),
        pltpu.VMEM((EPW,), jnp.float32),
        pltpu.VMEM((CHUNK,), jnp.float32),
        pltpu.VMEM((ZSTRIPE,), jnp.float32),
        pltpu.VMEM_SHARED((NACC,), jnp.float32),
        pltpu.VMEM_SHARED((NACC,), jnp.float32),
        pltpu.VMEM((C,), jnp.int32),
        pltpu.VMEM((C,), jnp.float32),
        pltpu.SemaphoreType.DMA,
        pltpu.SemaphoreType.DMA,
    ],
)


def _final_body(p_ref, out_ref):
    p = p_ref[...]
    loss_sum = p[0:1, :] + p[2:3, :]
    cnt = p[1:2, :] + p[3:4, :]
    aux = jnp.sum(loss_sum / jnp.maximum(cnt, 1.0)) * (1.0 / C)
    pairs = jnp.sum(cnt)
    mpl = jnp.sum(loss_sum) / pairs
    mdeg = jnp.max(cnt)
    lane = lax.broadcasted_iota(jnp.int32, (1, C), 1)
    row = jnp.where(lane == 0, aux,
                    jnp.where(lane == 1, pairs,
                              jnp.where(lane == 2, mpl, mdeg)))
    out_ref[...] = row


def _finalize(partials):
    return pl.pallas_call(
        _final_body,
        out_shape=jax.ShapeDtypeStruct((1, C), jnp.float32),
    )(partials.reshape(2 * 2, C))


def kernel(H, edge_index, centers, target, W, b):
    H = H.astype(jnp.float32)
    target = target.astype(jnp.float32)
    W = W.astype(jnp.float32)
    b = b.astype(jnp.float32)
    edges = edge_index.astype(jnp.int32)
    centers = centers.astype(jnp.int32)

    err2d = _err_node(H, target, W, b)
    err_flat = err2d.reshape(N_PAD)

    src = edges[0]
    dst = edges[1]
    pad = E_PAD - E
    src_p = jnp.concatenate([src, jnp.full((pad,), N, jnp.int32)])
    dst_p = jnp.concatenate([dst, jnp.zeros((pad,), jnp.int32)])
    src3 = src_p.reshape(NW, NCHUNK, CHUNK)
    dst2 = dst_p.reshape(NW, EPW)

    partials = _sc_scatter()(src3, dst2, err_flat, centers)
    row = _finalize(partials)

    aux_loss = row[0, 0]
    stats_pairs = row[0, 1]
    mean_pair_loss = row[0, 2]
    max_deg = row[0, 3]
    stats_centers = jnp.asarray(float(C), dtype=jnp.float32)
    return (aux_loss, stats_centers, stats_pairs, mean_pair_loss, max_deg)
